# baseline (speedup 1.0000x reference)
import functools

import jax
import jax.numpy as jnp
from jax import lax
from jax.experimental import pallas as pl
from jax.experimental.pallas import tpu as pltpu
from jax.experimental.pallas import tpu_sc as plsc

GRID = 128
NF = 10
D = 2 * NF
OUTD = 6 * NF

NC = 2
NS = 16
L = 16
NW = NC * NS

CHUNK = 512

PC = [-1.0, 19.739206314086914, -64.93917083740234, 85.45116424560547,
      -60.17622375488281, 26.000497817993164, -6.575565814971924]
PS = [-6.2831854820251465, 41.34170150756836, -81.60515594482422,
      76.70345306396484, -42.029598236083984, 14.913905143737793,
      -3.258183240890503]


def _poly(coef, s):
    acc = jnp.full((L,), coef[-1], jnp.float32)
    for c in coef[-2::-1]:
        acc = acc * s + jnp.float32(c)
    return acc


@jax.jit
def _lpe_sc(x, table):
    B = x.shape[0]
    b_per_w = B // NW
    nchunks = b_per_w // CHUNK
    mesh = plsc.VectorSubcoreMesh(
        core_axis_name="c", subcore_axis_name="s",
        num_cores=NC, num_subcores=NS)

    @functools.partial(
        pl.kernel,
        out_type=jax.ShapeDtypeStruct((B, OUTD), jnp.float32),
        scratch_types=[
            pltpu.VMEM((CHUNK, 3), jnp.float32),
            pltpu.VMEM((CHUNK // 128, 128), jnp.int32),
            pltpu.VMEM((CHUNK, 2 * D), jnp.float32),
            pltpu.VMEM((CHUNK, OUTD), jnp.float32),
            pltpu.SemaphoreType.DMA,
        ],
        mesh=mesh,
        compiler_params=pltpu.CompilerParams(
            needs_layout_passes=False, use_tc_tiling_on_sc=False),
    )
    def sck(x_hbm, tab_hbm, out_hbm, x_v, idx_v, ape_v, out_v, sem):
        wid = lax.axis_index("s") * NC + lax.axis_index("c")
        base = wid * b_per_w
        lanes = lax.iota(jnp.int32, L)

        def chunk_body(ci, carry):
            p0 = base + ci * CHUNK
            pltpu.sync_copy(x_hbm.at[pl.ds(p0, CHUNK)], x_v)

            for g in range(CHUNK // L):
                gl = lanes + g * L
                comps = []
                for j in range(3):
                    xl = plsc.load_gather(
                        x_v, [gl, jnp.full((L,), j, jnp.int32)])
                    ii = (xl * jnp.float32(GRID)).astype(jnp.int32)
                    comps.append(jnp.clip(ii, 0, GRID - 1))
                flat = (comps[0] * GRID + comps[1]) * GRID + comps[2]
                idx_v[g // 8, pl.ds((g % 8) * L, L)] = (
                    lax.shift_right_logical(flat, 1))

            copies = [
                pltpu.async_copy(
                    tab_hbm.at[idx_v.at[k]],
                    ape_v.at[pl.ds(k * 128, 128)],
                    sem,
                )
                for k in range(CHUNK // 128)
            ]
            for cp in copies:
                cp.wait()

            def mod(g, carry2):
                gl = lanes + g * L
                xls = [
                    plsc.load_gather(
                        x_v, [gl, jnp.full((L,), j, jnp.int32)])
                    * jnp.float32(GRID)
                    for j in range(3)
                ]
                iis = [xl.astype(jnp.int32) for xl in xls]
                half = (jnp.clip(iis[2], 0, GRID - 1) & 1) * D
                gains = [
                    plsc.load_gather(ape_v, [gl, half + f])
                    for f in range(D)
                ]
                for j in range(3):
                    xl = xls[j]
                    fr = xl - iis[j].astype(jnp.float32)
                    z = fr - jnp.float32(0.5)
                    sq = z * z
                    c1 = _poly(PC, sq)
                    s1 = z * _poly(PS, sq)
                    two_c1 = c1 + c1
                    cp2 = jnp.full((L,), 1.0, jnp.float32)
                    sp2 = jnp.full((L,), 0.0, jnp.float32)
                    cc, ss = c1, s1
                    for f in range(NF):
                        if f >= 1:
                            cn = two_c1 * cc - cp2
                            sn = two_c1 * ss - sp2
                            cp2, sp2, cc, ss = cc, ss, cn, sn
                        plsc.store_scatter(
                            out_v, [gl, jnp.full((L,), 6 * f + j, jnp.int32)],
                            cc * gains[f])
                        plsc.store_scatter(
                            out_v,
                            [gl, jnp.full((L,), 6 * f + 3 + j, jnp.int32)],
                            ss * gains[f + NF])
                return carry2

            lax.fori_loop(0, CHUNK // L, mod, 0)

            pltpu.sync_copy(out_v, out_hbm.at[pl.ds(p0, CHUNK)])
            return carry

        lax.fori_loop(0, nchunks, chunk_body, 0)

    return sck(x, table)


def kernel(x, latent_grid):
    table = latent_grid.reshape(-1, 2 * latent_grid.shape[-1])
    return _lpe_sc(x, table)


# speedup vs baseline: 1.1268x; 1.1268x over previous
import functools

import jax
import jax.numpy as jnp
from jax import lax
from jax.experimental import pallas as pl
from jax.experimental.pallas import tpu as pltpu
from jax.experimental.pallas import tpu_sc as plsc

GRID = 128
NF = 10
D = 2 * NF
OUTD = 6 * NF

NC = 2
NS = 16
L = 16
NW = NC * NS

CHUNK = 512

PC = [-1.0, 19.739206314086914, -64.93917083740234, 85.45116424560547,
      -60.17622375488281, 26.000497817993164, -6.575565814971924]
PS = [-6.2831854820251465, 41.34170150756836, -81.60515594482422,
      76.70345306396484, -42.029598236083984, 14.913905143737793,
      -3.258183240890503]


def _poly(coef, s):
    acc = jnp.full((L,), coef[-1], jnp.float32)
    for c in coef[-2::-1]:
        acc = acc * s + jnp.float32(c)
    return acc


@jax.jit
def _lpe_sc(x0, x1, x2, table):
    B = x0.shape[0]
    b_per_w = B // NW
    nchunks = b_per_w // CHUNK
    mesh = plsc.VectorSubcoreMesh(
        core_axis_name="c", subcore_axis_name="s",
        num_cores=NC, num_subcores=NS)

    @functools.partial(
        pl.kernel,
        out_type=jax.ShapeDtypeStruct((B * OUTD,), jnp.float32),
        scratch_types=[
            pltpu.VMEM((CHUNK,), jnp.float32),
            pltpu.VMEM((CHUNK,), jnp.float32),
            pltpu.VMEM((CHUNK,), jnp.float32),
            pltpu.VMEM((CHUNK // 128, 128), jnp.int32),
            pltpu.VMEM((CHUNK, 2 * D), jnp.float32),
            pltpu.VMEM((CHUNK * OUTD,), jnp.float32),
            pltpu.SemaphoreType.DMA,
        ],
        mesh=mesh,
        compiler_params=pltpu.CompilerParams(
            needs_layout_passes=False, use_tc_tiling_on_sc=False),
    )
    def sck(x0_hbm, x1_hbm, x2_hbm, tab_hbm, out_hbm,
            x0_v, x1_v, x2_v, idx_v, ape_v, out_v, sem):
        wid = lax.axis_index("s") * NC + lax.axis_index("c")
        base = wid * b_per_w
        lanes = lax.iota(jnp.int32, L)
        xvs = (x0_v, x1_v, x2_v)

        def chunk_body(ci, carry):
            p0 = base + ci * CHUNK
            pltpu.sync_copy(x0_hbm.at[pl.ds(p0, CHUNK)], x0_v)
            pltpu.sync_copy(x1_hbm.at[pl.ds(p0, CHUNK)], x1_v)
            pltpu.sync_copy(x2_hbm.at[pl.ds(p0, CHUNK)], x2_v)

            for g in range(CHUNK // L):
                comps = []
                for j in range(3):
                    xl = xvs[j][pl.ds(g * L, L)] * jnp.float32(GRID)
                    ii = xl.astype(jnp.int32)
                    comps.append(jnp.clip(ii, 0, GRID - 1))
                flat = (comps[0] * GRID + comps[1]) * GRID + comps[2]
                idx_v[g // 8, pl.ds((g % 8) * L, L)] = (
                    lax.shift_right_logical(flat, 1))

            copies = [
                pltpu.async_copy(
                    tab_hbm.at[idx_v.at[k]],
                    ape_v.at[pl.ds(k * 128, 128)],
                    sem,
                )
                for k in range(CHUNK // 128)
            ]
            for cp in copies:
                cp.wait()

            def mod(g, carry2):
                gl = lanes + g * L
                out_base = gl * OUTD
                xls = [xvs[j][pl.ds(g * L, L)] * jnp.float32(GRID)
                       for j in range(3)]
                iis = [xl.astype(jnp.int32) for xl in xls]
                half = (jnp.clip(iis[2], 0, GRID - 1) & 1) * D
                gains = [
                    plsc.load_gather(ape_v, [gl, half + f])
                    for f in range(D)
                ]
                for j in range(3):
                    xl = xls[j]
                    fr = xl - iis[j].astype(jnp.float32)
                    z = fr - jnp.float32(0.5)
                    sq = z * z
                    c1 = _poly(PC, sq)
                    s1 = z * _poly(PS, sq)
                    two_c1 = c1 + c1
                    cp2 = jnp.full((L,), 1.0, jnp.float32)
                    sp2 = jnp.full((L,), 0.0, jnp.float32)
                    cc, ss = c1, s1
                    for f in range(NF):
                        if f >= 1:
                            cn = two_c1 * cc - cp2
                            sn = two_c1 * ss - sp2
                            cp2, sp2, cc, ss = cc, ss, cn, sn
                        plsc.store_scatter(
                            out_v, [out_base + (6 * f + j)], cc * gains[f])
                        plsc.store_scatter(
                            out_v, [out_base + (6 * f + 3 + j)],
                            ss * gains[f + NF])
                return carry2

            lax.fori_loop(0, CHUNK // L, mod, 0)

            pltpu.sync_copy(out_v, out_hbm.at[pl.ds(p0 * OUTD, CHUNK * OUTD)])
            return carry

        lax.fori_loop(0, nchunks, chunk_body, 0)

    return sck(x0, x1, x2, table)


def kernel(x, latent_grid):
    B = x.shape[0]
    table = latent_grid.reshape(-1, 2 * latent_grid.shape[-1])
    out = _lpe_sc(x[:, 0], x[:, 1], x[:, 2], table)
    return out.reshape(B, OUTD)


# speedup vs baseline: 1.3285x; 1.1790x over previous
import functools

import jax
import jax.numpy as jnp
from jax import lax
from jax.experimental import pallas as pl
from jax.experimental.pallas import tpu as pltpu
from jax.experimental.pallas import tpu_sc as plsc

GRID = 128
NF = 10
D = 2 * NF
OUTD = 6 * NF

NC = 2
NS = 16
L = 16
NW = NC * NS

CHUNK = 512

PC = [-1.0, 19.739206314086914, -64.93917083740234, 85.45116424560547,
      -60.17622375488281, 26.000497817993164, -6.575565814971924]
PS = [-6.2831854820251465, 41.34170150756836, -81.60515594482422,
      76.70345306396484, -42.029598236083984, 14.913905143737793,
      -3.258183240890503]


def _poly(coef, s):
    acc = jnp.full((L,), coef[-1], jnp.float32)
    for c in coef[-2::-1]:
        acc = acc * s + jnp.float32(c)
    return acc


@jax.jit
def _lpe_sc(x0, x1, x2, table):
    B = x0.shape[0]
    b_per_w = B // NW
    nchunks = b_per_w // CHUNK
    mesh = plsc.VectorSubcoreMesh(
        core_axis_name="c", subcore_axis_name="s",
        num_cores=NC, num_subcores=NS)

    @functools.partial(
        pl.kernel,
        out_type=jax.ShapeDtypeStruct((8 * (B // 128) * 8 * 128,), jnp.float32),
        scratch_types=[
            pltpu.VMEM((CHUNK,), jnp.float32),
            pltpu.VMEM((CHUNK,), jnp.float32),
            pltpu.VMEM((CHUNK,), jnp.float32),
            pltpu.VMEM((CHUNK // 128, 128), jnp.int32),
            pltpu.VMEM((CHUNK, 2 * D), jnp.float32),
            pltpu.VMEM((8 * (CHUNK // 128) * 8 * 128,), jnp.float32),
            pltpu.SemaphoreType.DMA,
        ],
        mesh=mesh,
        compiler_params=pltpu.CompilerParams(
            needs_layout_passes=False, use_tc_tiling_on_sc=False),
    )
    def sck(x0_hbm, x1_hbm, x2_hbm, tab_hbm, out_hbm,
            x0_v, x1_v, x2_v, idx_v, ape_v, out_v, sem):
        wid = lax.axis_index("s") * NC + lax.axis_index("c")
        base = wid * b_per_w
        lanes = lax.iota(jnp.int32, L)
        xvs = (x0_v, x1_v, x2_v)

        def chunk_body(ci, carry):
            p0 = base + ci * CHUNK
            pltpu.sync_copy(x0_hbm.at[pl.ds(p0, CHUNK)], x0_v)
            pltpu.sync_copy(x1_hbm.at[pl.ds(p0, CHUNK)], x1_v)
            pltpu.sync_copy(x2_hbm.at[pl.ds(p0, CHUNK)], x2_v)

            for g in range(CHUNK // L):
                comps = []
                for j in range(3):
                    xl = xvs[j][pl.ds(g * L, L)] * jnp.float32(GRID)
                    ii = xl.astype(jnp.int32)
                    comps.append(jnp.clip(ii, 0, GRID - 1))
                flat = (comps[0] * GRID + comps[1]) * GRID + comps[2]
                idx_v[g // 8, pl.ds((g % 8) * L, L)] = (
                    lax.shift_right_logical(flat, 1))

            copies = [
                pltpu.async_copy(
                    tab_hbm.at[idx_v.at[k]],
                    ape_v.at[pl.ds(k * 128, 128)],
                    sem,
                )
                for k in range(CHUNK // 128)
            ]
            for cp in copies:
                cp.wait()

            def mod(g, carry2):
                gl = lanes + g * L
                tile_off = (g // 8) * 1024 + (g % 8) * L
                xls = [xvs[j][pl.ds(g * L, L)] * jnp.float32(GRID)
                       for j in range(3)]
                iis = [xl.astype(jnp.int32) for xl in xls]
                half = (jnp.clip(iis[2], 0, GRID - 1) & 1) * D
                gains = [
                    plsc.load_gather(ape_v, [gl, half + f])
                    for f in range(D)
                ]
                for j in range(3):
                    xl = xls[j]
                    fr = xl - iis[j].astype(jnp.float32)
                    z = fr - jnp.float32(0.5)
                    sq = z * z
                    c1 = _poly(PC, sq)
                    s1 = z * _poly(PS, sq)
                    two_c1 = c1 + c1
                    cp2 = jnp.full((L,), 1.0, jnp.float32)
                    sp2 = jnp.full((L,), 0.0, jnp.float32)
                    cc, ss = c1, s1
                    for f in range(NF):
                        if f >= 1:
                            cn = two_c1 * cc - cp2
                            sn = two_c1 * ss - sp2
                            cp2, sp2, cc, ss = cc, ss, cn, sn
                        c_cos = 6 * f + j
                        c_sin = 6 * f + 3 + j
                        out_v[pl.ds((c_cos // 8) * 4096 + (c_cos % 8) * 128
                                    + tile_off, L)] = cc * gains[f]
                        out_v[pl.ds((c_sin // 8) * 4096 + (c_sin % 8) * 128
                                    + tile_off, L)] = ss * gains[f + NF]
                return carry2

            lax.fori_loop(0, CHUNK // L, mod, 0)

            pt0 = p0 // 128
            for a in range(8):
                pltpu.sync_copy(
                    out_v.at[pl.ds(a * 4096, 4096)],
                    out_hbm.at[pl.ds(a * ((B // 128) * 1024) + pt0 * 1024,
                                     4096)],
                )
            return carry

        lax.fori_loop(0, nchunks, chunk_body, 0)

    return sck(x0, x1, x2, table)


def kernel(x, latent_grid):
    B = x.shape[0]
    table = latent_grid.reshape(-1, 2 * latent_grid.shape[-1])
    flat = _lpe_sc(x[:, 0], x[:, 1], x[:, 2], table)
    out4 = flat.reshape(8, B // 128, 8, 128)
    return out4.transpose(1, 3, 0, 2).reshape(B, 64)[:, :OUTD]
